# Initial kernel scaffold; baseline (speedup 1.0000x reference)
import jax
import jax.numpy as jnp
from jax.experimental import pallas as pl


def kernel(x, h, edge_attr, edge_attr_partial, edge_index, partial_goal_mask, params, eps):
    raise NotImplementedError("write your pallas kernel here")


# speedup vs baseline: 5.4646x; 5.4646x over previous
import functools

import jax
import jax.numpy as jnp
from jax import lax
from jax.experimental import pallas as pl
from jax.experimental.pallas import tpu as pltpu
from jax.experimental.pallas import tpu_sc as plsc

N = 10000
E = 320000
BN = 1000
BE = 2000
N_TILES = 16
N_WORKERS = 32
DRB = 80
NDB = N // DRB


def _relu(v):
    return jnp.maximum(v, 0.0)


def _softplus(v):
    return jnp.maximum(v, 0.0) + jnp.log(1.0 + jnp.exp(-jnp.abs(v)))


def _rb(b, d):
    return pl.BlockSpec((b, d), lambda i: (i, 0))


def _fb(s0, s1):
    return pl.BlockSpec((s0, s1), lambda i: (0, 0))


def _tc_call(body, grid, in_specs, out_specs, out_shapes):
    return pl.pallas_call(
        body,
        grid=(grid,),
        in_specs=in_specs,
        out_specs=out_specs,
        out_shape=out_shapes,
        compiler_params=pltpu.CompilerParams(dimension_semantics=("parallel",)),
    )


def _tcA_body(x_r, h_r, m_r, Wxg, Whg, bg, Wxp, Whp, bp,
              Wsg, Wdg, bmg, Wsp, Wdp, bmp,
              hidg_o, hidp_o, A_o, B_o):
    xb = x_r[...]
    hb = h_r[...]
    mb = m_r[...]
    hg = _relu(jnp.dot(xb, Wxg[...]) + jnp.dot(hb, Whg[...]) + bg[...])
    hp = _relu(jnp.dot(xb * mb, Wxp[...]) + jnp.dot(hb, Whp[...]) + bp[...])
    hidg_o[...] = hg
    hidp_o[...] = hp
    A_o[...] = jnp.concatenate([jnp.dot(hg, Wsg[...]), jnp.dot(hp, Wsp[...])], axis=1)
    B_o[...] = jnp.concatenate([jnp.dot(hg, Wdg[...]) + bmg[...],
                                jnp.dot(hp, Wdp[...]) + bmp[...]], axis=1)


def _tcB_body(ea_r, eap_r, Weg, beg, Wep, bep, Wg0, Wg1, Wp0, Wp1, C0_o, C1_o):
    eg = _relu(jnp.dot(ea_r[...], Weg[...]) + beg[...])
    ep = _relu(jnp.dot(eap_r[...], Wep[...]) + bep[...])
    C0_o[...] = jnp.concatenate([jnp.dot(eg, Wg0[...]), jnp.dot(ep, Wp0[...])], axis=1)
    C1_o[...] = jnp.concatenate([jnp.dot(eg, Wg1[...]), jnp.dot(ep, Wp1[...])], axis=1)


def _tcC_body(hg_r, hp_r, a0_r, a1_r,
              Wu1g, Wu2g, bug, Wu1p, Wu2p, bup,
              Wsg, Wdg, bmg, Wsp, Wdp, bmp,
              hg_o, hp_o, A_o, B_o):
    s = a0_r[...] + a1_r[...]
    hg = _relu(jnp.dot(hg_r[...], Wu1g[...]) + jnp.dot(s[:, :64], Wu2g[...]) + bug[...])
    hp = _relu(jnp.dot(hp_r[...], Wu1p[...]) + jnp.dot(s[:, 64:], Wu2p[...]) + bup[...])
    hg_o[...] = hg
    hp_o[...] = hp
    A_o[...] = jnp.concatenate([jnp.dot(hg, Wsg[...]), jnp.dot(hp, Wsp[...])], axis=1)
    B_o[...] = jnp.concatenate([jnp.dot(hg, Wdg[...]) + bmg[...],
                                jnp.dot(hp, Wdp[...]) + bmp[...]], axis=1)


def _tcD_body(hg_r, hp_r, a0_r, a1_r, h_r, eps_r,
              Wu1g, Wu2g, bug, Wu1p, Wu2p, bup,
              Woutg, boutg, Woutp, boutp,
              Wp1, bp1, Wp2, bp2, Wi1a, Wi1b, bi1, Wi2, bi2,
              Wz, Wgp, Whd, bind, Wsd, Wdd, bmd, bed, Wmed0,
              locq_o, scaleq_o, locp_o, scalep_o, hidd_o, A_o, B_o):
    s = a0_r[...] + a1_r[...]
    hg = _relu(jnp.dot(hg_r[...], Wu1g[...]) + jnp.dot(s[:, :64], Wu2g[...]) + bug[...])
    hp = _relu(jnp.dot(hp_r[...], Wu1p[...]) + jnp.dot(s[:, 64:], Wu2p[...]) + bup[...])
    z_goal = jnp.dot(hg, Woutg[...]) + boutg[...]
    z_gp = jnp.dot(hp, Woutp[...]) + boutp[...]
    tp = _relu(jnp.dot(z_gp, Wp1[...]) + bp1[...])
    op = jnp.dot(tp, Wp2[...]) + bp2[...]
    locp_o[...] = op[:, :64]
    scalep_o[...] = _softplus(op[:, 64:]) + 1e-4
    ti = _relu(jnp.dot(z_goal, Wi1a[...]) + jnp.dot(z_gp, Wi1b[...]) + bi1[...])
    oi = jnp.dot(ti, Wi2[...]) + bi2[...]
    locq = oi[:, :64]
    scaleq = _softplus(oi[:, 64:]) + 1e-4
    locq_o[...] = locq
    scaleq_o[...] = scaleq
    z = locq + scaleq * eps_r[...]
    hd = _relu(jnp.dot(z, Wz[...]) + jnp.dot(z_gp, Wgp[...])
               + jnp.dot(h_r[...], Whd[...]) + bind[...])
    hidd_o[...] = hd
    c0 = jnp.dot(_relu(bed[...]), Wmed0[...])
    A_o[...] = jnp.dot(hd, Wsd[...])
    B_o[...] = jnp.dot(hd, Wdd[...]) + bmd[...] + c0


def _tcE_body(hd_r, a0_r, a1_r, Wu1, Wu2, bu, Wsd, Wdd, bmd, bed, Wmed1,
              hd_o, A_o, B_o):
    s = a0_r[...] + a1_r[...]
    hd = _relu(jnp.dot(hd_r[...], Wu1[...]) + jnp.dot(s, Wu2[...]) + bu[...])
    hd_o[...] = hd
    c1 = jnp.dot(_relu(bed[...]), Wmed1[...])
    A_o[...] = jnp.dot(hd, Wsd[...])
    B_o[...] = jnp.dot(hd, Wdd[...]) + bmd[...] + c1


def _tcF_body(hd_r, a0_r, a1_r, Wu1, Wu2, bu, Wout, bout, mu_o):
    s = a0_r[...] + a1_r[...]
    hd = _relu(jnp.dot(hd_r[...], Wu1[...]) + jnp.dot(s, Wu2[...]) + bu[...])
    mu_o[...] = jnp.dot(hd, Wout[...]) + bout[...]


def _make_sc_edge(W, with_c):
    K = 128 if with_c else 512
    SUB = K // 128
    NCH = E // K
    mesh = plsc.VectorSubcoreMesh(core_axis_name="c", subcore_axis_name="s")

    scratch = [
        pltpu.VMEM((SUB, 128), jnp.int32),
        pltpu.VMEM((SUB, 128), jnp.int32),
        pltpu.VMEM((K, W), jnp.float32),
        pltpu.VMEM((K, W), jnp.float32),
    ]
    if with_c:
        scratch.append(pltpu.VMEM((K, W), jnp.float32))
    scratch.append(pltpu.VMEM_SHARED((N, W), jnp.float32))
    scratch.append(pltpu.SemaphoreType.DMA)

    def body(*refs):
        if with_c:
            (a_hbm, b_hbm, c_hbm, src_hbm, dst_hbm, z_hbm, out_hbm,
             idx_s, idx_d, a_buf, b_buf, c_buf, agg, sem) = refs
        else:
            (a_hbm, b_hbm, src_hbm, dst_hbm, z_hbm, out_hbm,
             idx_s, idx_d, a_buf, b_buf, agg, sem) = refs
        cid = lax.axis_index("c")
        sid = lax.axis_index("s")
        wid = sid * 2 + cid
        ndb = (NDB + (N_TILES - 1) - sid) // N_TILES

        def zero_blk(i, cc):
            r0 = (sid + i * N_TILES) * DRB
            pltpu.sync_copy(z_hbm.at[pl.ds(r0, DRB)], agg.at[pl.ds(r0, DRB)])
            return cc
        lax.fori_loop(0, ndb, zero_blk, 0)
        plsc.subcore_barrier()

        def chunk(i, carry):
            g = wid + i * N_WORKERS
            base = g * K
            for j in range(SUB):
                pltpu.sync_copy(src_hbm.at[pl.ds(base + j * 128, 128)], idx_s.at[j])
                pltpu.sync_copy(dst_hbm.at[pl.ds(base + j * 128, 128)], idx_d.at[j])
            descs = []
            for j in range(SUB):
                descs.append(pltpu.async_copy(
                    a_hbm.at[idx_s.at[j]], a_buf.at[pl.ds(j * 128, 128)], sem))
                descs.append(pltpu.async_copy(
                    b_hbm.at[idx_d.at[j]], b_buf.at[pl.ds(j * 128, 128)], sem))
            if with_c:
                pltpu.sync_copy(c_hbm.at[pl.ds(base, K)], c_buf)
            for d in descs:
                d.wait()

            def row(r, cc):
                for j in range(W // 16):
                    sl = pl.ds(j * 16, 16)
                    v = a_buf[r, sl] + b_buf[r, sl]
                    if with_c:
                        v = v + c_buf[r, sl]
                    a_buf[r, sl] = jnp.maximum(v, 0.0)
                return cc
            lax.fori_loop(0, K, row, 0)

            for j in range(SUB):
                pltpu.sync_copy(a_buf.at[pl.ds(j * 128, 128)],
                                agg.at[idx_d.at[j]], add=True)
            return carry

        nc = (NCH + (N_WORKERS - 1) - wid) // N_WORKERS
        lax.fori_loop(0, nc, chunk, 0)
        plsc.subcore_barrier()

        def drain_blk(i, cc):
            r0 = (sid + i * N_TILES) * DRB
            pltpu.sync_copy(agg.at[pl.ds(r0, DRB)],
                            out_hbm.at[cid, pl.ds(r0, DRB)])
            return cc
        lax.fori_loop(0, ndb, drain_blk, 0)

    return pl.kernel(
        body,
        out_type=jax.ShapeDtypeStruct((2, N, W), jnp.float32),
        mesh=mesh,
        scratch_types=scratch,
        compiler_params=pltpu.CompilerParams(use_tc_tiling_on_sc=(W == 128)),
    )


_SC_EDGE_C = _make_sc_edge(128, True)
_SC_EDGE_NC = _make_sc_edge(64, False)


def kernel(x, h, edge_attr, edge_attr_partial, edge_index, partial_goal_mask, params, eps):
    src = edge_index[0]
    dst = edge_index[1]
    m2 = partial_goal_mask[:, None]
    pg, pp, pd = params["goal"], params["partial"], params["dec"]
    pr, pi = params["prior"], params["inf"]

    def r1(v):
        return v.reshape(1, -1)

    def mparts(lp):
        Wm = lp["W_m"]
        return Wm[:64], Wm[64:128], Wm[128:], r1(lp["b_m"])

    def uparts(lp):
        Wu = lp["W_u"]
        return Wu[:64], Wu[64:], r1(lp["b_u"])

    zeros128 = jnp.zeros((N, 128), jnp.float32)
    zeros64 = jnp.zeros((N, 64), jnp.float32)

    Wsg0, Wdg0, Wmeg0, bmg0 = mparts(pg["layers"][0])
    Wsp0, Wdp0, Wmep0, bmp0 = mparts(pp["layers"][0])
    w64 = [(BN, 64)] * 2
    hidg, hidp, A0, B0 = _tc_call(
        _tcA_body, N // BN,
        [_rb(BN, 3), _rb(BN, 16), _rb(BN, 1),
         _fb(3, 64), _fb(16, 64), _fb(1, 64),
         _fb(3, 64), _fb(16, 64), _fb(1, 64),
         _fb(64, 64), _fb(64, 64), _fb(1, 64),
         _fb(64, 64), _fb(64, 64), _fb(1, 64)],
        [_rb(BN, 64), _rb(BN, 64), _rb(BN, 128), _rb(BN, 128)],
        [jax.ShapeDtypeStruct((N, 64), jnp.float32)] * 2
        + [jax.ShapeDtypeStruct((N, 128), jnp.float32)] * 2,
    )(x, h, m2,
      pg["W_in"][:3], pg["W_in"][3:], r1(pg["b_in"]),
      pp["W_in"][:3], pp["W_in"][3:], r1(pp["b_in"]),
      Wsg0, Wdg0, bmg0, Wsp0, Wdp0, bmp0)

    Wsg1, Wdg1, Wmeg1, bmg1 = mparts(pg["layers"][1])
    Wsp1, Wdp1, Wmep1, bmp1 = mparts(pp["layers"][1])
    C0, C1 = _tc_call(
        _tcB_body, E // BE,
        [_rb(BE, 4), _rb(BE, 4),
         _fb(4, 64), _fb(1, 64), _fb(4, 64), _fb(1, 64),
         _fb(64, 64), _fb(64, 64), _fb(64, 64), _fb(64, 64)],
        [_rb(BE, 128), _rb(BE, 128)],
        [jax.ShapeDtypeStruct((E, 128), jnp.float32)] * 2,
    )(edge_attr, edge_attr_partial,
      pg["W_e"], r1(pg["b_e"]), pp["W_e"], r1(pp["b_e"]),
      Wmeg0, Wmeg1, Wmep0, Wmep1)

    agg0 = _SC_EDGE_C(A0, B0, C0, src, dst, zeros128)

    Wu1g0, Wu2g0, bug0 = uparts(pg["layers"][0])
    Wu1p0, Wu2p0, bup0 = uparts(pp["layers"][0])
    hidg1, hidp1, A1, B1 = _tc_call(
        _tcC_body, N // BN,
        [_rb(BN, 64), _rb(BN, 64), _rb(BN, 128), _rb(BN, 128),
         _fb(64, 64), _fb(64, 64), _fb(1, 64),
         _fb(64, 64), _fb(64, 64), _fb(1, 64),
         _fb(64, 64), _fb(64, 64), _fb(1, 64),
         _fb(64, 64), _fb(64, 64), _fb(1, 64)],
        [_rb(BN, 64), _rb(BN, 64), _rb(BN, 128), _rb(BN, 128)],
        [jax.ShapeDtypeStruct((N, 64), jnp.float32)] * 2
        + [jax.ShapeDtypeStruct((N, 128), jnp.float32)] * 2,
    )(hidg, hidp, agg0[0], agg0[1],
      Wu1g0, Wu2g0, bug0, Wu1p0, Wu2p0, bup0,
      Wsg1, Wdg1, bmg1, Wsp1, Wdp1, bmp1)

    agg1 = _SC_EDGE_C(A1, B1, C1, src, dst, zeros128)

    Wu1g1, Wu2g1, bug1 = uparts(pg["layers"][1])
    Wu1p1, Wu2p1, bup1 = uparts(pp["layers"][1])
    Wsd0, Wdd0, Wmed0, bmd0 = mparts(pd["layers"][0])
    locq, scaleq, locp, scalep, hidd0, Ad0, Bd0 = _tc_call(
        _tcD_body, N // BN,
        [_rb(BN, 64), _rb(BN, 64), _rb(BN, 128), _rb(BN, 128),
         _rb(BN, 16), _rb(BN, 64),
         _fb(64, 64), _fb(64, 64), _fb(1, 64),
         _fb(64, 64), _fb(64, 64), _fb(1, 64),
         _fb(64, 64), _fb(1, 64), _fb(64, 64), _fb(1, 64),
         _fb(64, 128), _fb(1, 128), _fb(128, 128), _fb(1, 128),
         _fb(64, 128), _fb(64, 128), _fb(1, 128), _fb(128, 128), _fb(1, 128),
         _fb(64, 64), _fb(64, 64), _fb(16, 64), _fb(1, 64),
         _fb(64, 64), _fb(64, 64), _fb(1, 64), _fb(1, 64), _fb(64, 64)],
        [_rb(BN, 64)] * 7,
        [jax.ShapeDtypeStruct((N, 64), jnp.float32)] * 7,
    )(hidg1, hidp1, agg1[0], agg1[1], h, eps,
      Wu1g1, Wu2g1, bug1, Wu1p1, Wu2p1, bup1,
      pg["W_out"], r1(pg["b_out"]), pp["W_out"], r1(pp["b_out"]),
      pr["W1"], r1(pr["b1"]), pr["W2"], r1(pr["b2"]),
      pi["W1"][:64], pi["W1"][64:], r1(pi["b1"]), pi["W2"], r1(pi["b2"]),
      pd["W_in"][:64], pd["W_in"][64:128], pd["W_in"][128:], r1(pd["b_in"]),
      Wsd0, Wdd0, bmd0, r1(pd["b_e"]), Wmed0)

    aggd0 = _SC_EDGE_NC(Ad0, Bd0, src, dst, zeros64)

    Wu1d0, Wu2d0, bud0 = uparts(pd["layers"][0])
    Wsd1, Wdd1, Wmed1, bmd1 = mparts(pd["layers"][1])
    hidd1, Ad1, Bd1 = _tc_call(
        _tcE_body, N // BN,
        [_rb(BN, 64)] * 3
        + [_fb(64, 64), _fb(64, 64), _fb(1, 64),
           _fb(64, 64), _fb(64, 64), _fb(1, 64), _fb(1, 64), _fb(64, 64)],
        [_rb(BN, 64)] * 3,
        [jax.ShapeDtypeStruct((N, 64), jnp.float32)] * 3,
    )(hidd0, aggd0[0], aggd0[1],
      Wu1d0, Wu2d0, bud0, Wsd1, Wdd1, bmd1, r1(pd["b_e"]), Wmed1)

    aggd1 = _SC_EDGE_NC(Ad1, Bd1, src, dst, zeros64)

    Wu1d1, Wu2d1, bud1 = uparts(pd["layers"][1])
    Wout_pad = jnp.pad(pd["W_out"], ((0, 0), (0, 125)))
    bout_pad = jnp.pad(r1(pd["b_out"]), ((0, 0), (0, 125)))
    (mu_pad,) = _tc_call(
        _tcF_body, N // BN,
        [_rb(BN, 64)] * 3
        + [_fb(64, 64), _fb(64, 64), _fb(1, 64), _fb(64, 128), _fb(1, 128)],
        [_rb(BN, 128)],
        [jax.ShapeDtypeStruct((N, 128), jnp.float32)],
    )(hidd1, aggd1[0], aggd1[1], Wu1d1, Wu2d1, bud1, Wout_pad, bout_pad)

    return mu_pad[:, :3], locq, scaleq, locp, scalep
